# Initial kernel scaffold; baseline (speedup 1.0000x reference)
import functools

import jax
import jax.numpy as jnp
from jax import lax
from jax.experimental import pallas as pl
from jax.experimental.pallas import tpu as pltpu
from jax.experimental.pallas import tpu_sc as plsc

N = 100000
EMBED = 128

_info = plsc.get_sparse_core_info()
NC, NS, L = _info.num_cores, _info.num_subcores, _info.num_lanes
NW = NC * NS

RPW = 3136
PAD_N = NW * RPW
G = 64
C = 448
NG = C // G
NCHUNK = RPW // C
TAIL = N - (PAD_N - C)


def _body(tids_hbm, nids_hbm, ttab_hbm, ntab_hbm, out_hbm,
          tidx, nidx, tbuf, nbuf, sem):
  wid = lax.axis_index("s") * NC + lax.axis_index("c")

  def chunk_body(chunk, carry):
    row0 = wid * RPW + chunk * C
    g0 = row0 // G

    pltpu.sync_copy(tids_hbm.at[pl.ds(g0, NG)], tidx)
    pltpu.sync_copy(nids_hbm.at[pl.ds(g0, NG)], nidx)

    copies = []
    for j in range(NG):
      copies.append(pltpu.async_copy(
          ttab_hbm.at[tidx.at[j]], tbuf.at[pl.ds(j * G, G)], sem))
    for j in range(NG):
      copies.append(pltpu.async_copy(
          ntab_hbm.at[nidx.at[j]], nbuf.at[pl.ds(j * G, G)], sem))
    for cp in copies:
      cp.wait()

    def add_row(r, c2):
      for c in range(EMBED // L):
        plsc.addupdate(tbuf.at[r, pl.ds(c * L, L)], nbuf[r, pl.ds(c * L, L)])
      return c2
    lax.fori_loop(0, C, add_row, 0)

    is_partial = row0 + C > N

    @pl.when(jnp.logical_not(is_partial))
    def _():
      pltpu.sync_copy(tbuf, out_hbm.at[pl.ds(row0, C)])

    @pl.when(is_partial)
    def _():
      pltpu.sync_copy(tbuf.at[pl.ds(0, TAIL)], out_hbm.at[pl.ds(row0, TAIL)])

    return carry

  lax.fori_loop(0, NCHUNK, chunk_body, 0)


_mesh = plsc.VectorSubcoreMesh(core_axis_name="c", subcore_axis_name="s")

_sc_embed = pl.kernel(
    _body,
    out_type=jax.ShapeDtypeStruct((N, EMBED), jnp.float32),
    mesh=_mesh,
    scratch_types=[
        pltpu.VMEM((NG, G), jnp.int32),
        pltpu.VMEM((NG, G), jnp.int32),
        pltpu.VMEM((C, EMBED), jnp.float32),
        pltpu.VMEM((C, EMBED), jnp.float32),
        pltpu.SemaphoreType.DMA,
    ],
)


@jax.jit
def kernel(token_ids, node_ids, token_table, node_table):
  pad = PAD_N - N
  tids = jnp.concatenate(
      [token_ids.astype(jnp.int32), jnp.zeros((pad,), jnp.int32)]
  ).reshape(PAD_N // G, G)
  nids = jnp.concatenate(
      [node_ids.astype(jnp.int32), jnp.zeros((pad,), jnp.int32)]
  ).reshape(PAD_N // G, G)
  return _sc_embed(tids, nids, token_table, node_table)


# speedup vs baseline: 1.7815x; 1.7815x over previous
import functools

import jax
import jax.numpy as jnp
from jax import lax
from jax.experimental import pallas as pl
from jax.experimental.pallas import tpu as pltpu
from jax.experimental.pallas import tpu_sc as plsc

N = 100000
EMBED = 128

_info = plsc.get_sparse_core_info()
NC, NS, L = _info.num_cores, _info.num_subcores, _info.num_lanes
NW = NC * NS

RPW = 3136
PAD_N = NW * RPW
G = 64
C = 448
NG = C // G
NCHUNK = RPW // C
TAIL = N - (PAD_N - C)


def _body(tids_hbm, nids_hbm, ttab_hbm, ntab_hbm, out_hbm,
          tidx, nidx, tbuf, nbuf, sem):
  wid = lax.axis_index("s") * NC + lax.axis_index("c")

  def chunk_body(chunk, carry):
    row0 = wid * RPW + chunk * C

    pltpu.sync_copy(tids_hbm.at[pl.ds(row0, C)], tidx)
    pltpu.sync_copy(nids_hbm.at[pl.ds(row0, C)], nidx)

    copies = []
    for j in range(NG):
      copies.append(pltpu.async_copy(
          ttab_hbm.at[tidx.at[pl.ds(j * G, G)]], tbuf.at[pl.ds(j * G, G)], sem))
    for j in range(NG):
      copies.append(pltpu.async_copy(
          ntab_hbm.at[nidx.at[pl.ds(j * G, G)]], nbuf.at[pl.ds(j * G, G)], sem))
    for cp in copies:
      cp.wait()

    def add_row(r, c2):
      for c in range(EMBED // L):
        plsc.addupdate(tbuf.at[r, pl.ds(c * L, L)], nbuf[r, pl.ds(c * L, L)])
      return c2
    lax.fori_loop(0, C, add_row, 0)

    is_partial = row0 + C > N

    @pl.when(jnp.logical_not(is_partial))
    def _():
      pltpu.sync_copy(tbuf, out_hbm.at[pl.ds(row0, C)])

    @pl.when(is_partial)
    def _():
      pltpu.sync_copy(tbuf.at[pl.ds(0, TAIL)], out_hbm.at[pl.ds(row0, TAIL)])

    return carry

  lax.fori_loop(0, NCHUNK, chunk_body, 0)


_mesh = plsc.VectorSubcoreMesh(core_axis_name="c", subcore_axis_name="s")

_sc_embed = pl.kernel(
    _body,
    out_type=jax.ShapeDtypeStruct((N, EMBED), jnp.float32),
    mesh=_mesh,
    scratch_types=[
        pltpu.VMEM((C,), jnp.int32),
        pltpu.VMEM((C,), jnp.int32),
        pltpu.VMEM((C, EMBED), jnp.float32),
        pltpu.VMEM((C, EMBED), jnp.float32),
        pltpu.SemaphoreType.DMA,
    ],
)


@jax.jit
def kernel(token_ids, node_ids, token_table, node_table):
  pad = PAD_N - N
  tids = jnp.concatenate(
      [token_ids.astype(jnp.int32), jnp.zeros((pad,), jnp.int32)])
  nids = jnp.concatenate(
      [node_ids.astype(jnp.int32), jnp.zeros((pad,), jnp.int32)])
  return _sc_embed(tids, nids, token_table, node_table)


# speedup vs baseline: 2.9912x; 1.6791x over previous
import jax
import jax.numpy as jnp
from jax import lax
from jax.experimental import pallas as pl
from jax.experimental.pallas import tpu as pltpu
from jax.experimental.pallas import tpu_sc as plsc

N = 100000
EMBED = 128

_info = plsc.get_sparse_core_info()
NC, NS, L = _info.num_cores, _info.num_subcores, _info.num_lanes
NW = NC * NS

RPW = 3136
C = 112
NCHUNK = RPW // C
NBUF = 4
IDX_BYTES = C * 4
ROW_BYTES = C * EMBED * 4


def _body(tids_hbm, nids_hbm, ttab_hbm, ntab_hbm, out_hbm, *scratch):
  tidx = scratch[0:NBUF]
  nidx = scratch[NBUF:2 * NBUF]
  tbuf = scratch[2 * NBUF:3 * NBUF]
  nbuf = scratch[3 * NBUF:4 * NBUF]
  isem = scratch[4 * NBUF:5 * NBUF]
  gsem = scratch[5 * NBUF:6 * NBUF]
  wsem = scratch[6 * NBUF:7 * NBUF]

  wid = lax.axis_index("s") * NC + lax.axis_index("c")
  base = jnp.minimum(wid * RPW, N - RPW)

  def issue_idx(chunk, b):
    row0 = base + chunk * C
    pltpu.async_copy(tids_hbm.at[pl.ds(row0, C)], tidx[b], isem[b])
    pltpu.async_copy(nids_hbm.at[pl.ds(row0, C)], nidx[b], isem[b])

  def wait_idx(b):
    pltpu.make_async_copy(tids_hbm.at[pl.ds(0, C)], tidx[b], isem[b]).wait()
    pltpu.make_async_copy(nids_hbm.at[pl.ds(0, C)], nidx[b], isem[b]).wait()

  def issue_gathers(b):
    pltpu.async_copy(ttab_hbm.at[tidx[b]], tbuf[b], gsem[b])
    pltpu.async_copy(ntab_hbm.at[nidx[b]], nbuf[b], gsem[b])

  def wait_gathers(b):
    pltpu.make_async_copy(ttab_hbm.at[pl.ds(0, C)], tbuf[b], gsem[b]).wait()
    pltpu.make_async_copy(ntab_hbm.at[pl.ds(0, C)], nbuf[b], gsem[b]).wait()

  def issue_write(chunk, b):
    row0 = base + chunk * C
    pltpu.async_copy(tbuf[b], out_hbm.at[pl.ds(row0, C)], wsem[b])

  def wait_write(b):
    pltpu.make_async_copy(tbuf[b], out_hbm.at[pl.ds(0, C)], wsem[b]).wait()

  def add_chunk(b):
    def add_row(r, carry):
      for c in range(EMBED // L):
        plsc.addupdate(tbuf[b].at[r, pl.ds(c * L, L)],
                       nbuf[b][r, pl.ds(c * L, L)])
      return carry
    lax.fori_loop(0, C, add_row, 0)

  for b in range(NBUF):
    issue_idx(b, b)
  for b in range(NBUF - 1):
    wait_idx(b)
    issue_gathers(b)

  def loop_body(k, carry):
    for s in range(NBUF):
      i = k * NBUF + s
      b = s
      bj = (s + NBUF - 1) % NBUF

      wait_gathers(b)
      add_chunk(b)
      issue_write(i, b)

      @pl.when(i + NBUF - 1 < NCHUNK)
      def _():
        @pl.when(i >= 1)
        def _():
          wait_write(bj)
        wait_idx(bj)
        issue_gathers(bj)

      @pl.when(i + NBUF < NCHUNK)
      def _():
        issue_idx(i + NBUF, b)
    return carry

  lax.fori_loop(0, NCHUNK // NBUF, loop_body, 0)

  for b in range(NBUF):
    wait_write(b)


_mesh = plsc.VectorSubcoreMesh(core_axis_name="c", subcore_axis_name="s")

_sc_embed = pl.kernel(
    _body,
    out_type=jax.ShapeDtypeStruct((N, EMBED), jnp.float32),
    mesh=_mesh,
    scratch_types=(
        [pltpu.VMEM((C,), jnp.int32) for _ in range(2 * NBUF)]
        + [pltpu.VMEM((C, EMBED), jnp.float32) for _ in range(2 * NBUF)]
        + [pltpu.SemaphoreType.DMA for _ in range(3 * NBUF)]
    ),
)


@jax.jit
def kernel(token_ids, node_ids, token_table, node_table):
  return _sc_embed(token_ids.astype(jnp.int32), node_ids.astype(jnp.int32),
                   token_table, node_table)
